# baseline (speedup 1.0000x reference)
import math

import jax
import jax.numpy as jnp
from jax.experimental import pallas as pl
from jax.experimental.pallas import tpu as pltpu

B = 16
S = 2048
H = 768
K = 8192
NSLOT = 9
STEPS = 4

S_T = 128
N_ST = S // S_T
K_T = 512
N_KT = K // K_T

_RSQRT_H = 1.0 / math.sqrt(float(H))


def _init_kernel(p_ref, l_ref, w_init_ref, state_ref, psum, lsum):
    i = pl.program_id(0)

    @pl.when(i == 0)
    def _():
        psum[:] = jnp.zeros_like(psum)
        lsum[:] = jnp.zeros_like(lsum)

    psum[:] += jnp.sum(p_ref[:], axis=1)
    lsum[:] += jnp.sum(l_ref[:], axis=1)

    @pl.when(i == N_ST - 1)
    def _():
        ps = psum[:] * (1.0 / S)
        ls = lsum[:] * (1.0 / S)
        cat = jnp.concatenate([ps, ls], axis=-1)
        state_ref[:] = jnp.tanh(_dot(cat, w_init_ref[:], ((1,), (1,))))


def _dot(a, b, dims):
    return jax.lax.dot_general(a, b, (dims, ((), ())),
                               preferred_element_type=jnp.float32,
                               precision=jax.lax.Precision.HIGHEST)


def _chain_body(state0_ref, slotq_ref, wq_ref, wk_ref, wv_ref, wop_ref,
                wsq_ref, wg_ref, bg_ref, ws1_ref, ws2_ref, bs_ref,
                wih_ref, whh_ref, bih_ref, bhh_ref,
                plo_ref, phi_ref, c_ref,
                stoplog_ref, stopprob_ref, summary_ref, chain_ref,
                state_s, q1k_s, ctxraw_s, ctx_s, oppre_s, q2k_s, g_s,
                raw2_s, bestv_s, beste_s, slog_s, sprob_s, gh_s):
    g = pl.program_id(0)
    p = g // B
    b = g % B
    s = p // 2
    is_a = (p % 2) == 0
    is_b = jnp.logical_not(is_a)

    @pl.when(g == 0)
    def _():
        state_s[:] = state0_ref[:]

    lo = plo_ref[0]
    hi = phi_ref[0]

    @pl.when(jnp.logical_and(is_a, b == 0))
    def _():
        sq = _dot(state_s[:], wq_ref[:], ((1,), (1,)))
        q1k_s[:] = _dot(sq, wk_ref[:], ((1,), (0,))) * _RSQRT_H

    @pl.when(jnp.logical_and(is_a, b < 3))
    def _():
        gh_s[:, pl.ds(b, 1), :] = _dot(state_s[:], whh_ref[:],
                                       ((1,), (1,)))[:, None, :]

    @pl.when(is_a)
    def _():
        q = q1k_s[pl.ds(b, 1), :]
        sc = jnp.concatenate(
            [_dot(q, lo, ((1,), (1,))), _dot(q, hi, ((1,), (1,)))], axis=1)
        m = jnp.max(sc, axis=1, keepdims=True)
        e = jnp.exp(sc - m)
        w = e / jnp.sum(e, axis=1, keepdims=True)
        ctxraw_s[pl.ds(b, 1), :] = (
            _dot(w[:, :S // 2], lo, ((1,), (0,)))
            + _dot(w[:, S // 2:], hi, ((1,), (0,))))

    @pl.when(jnp.logical_and(is_a, b == B - 1))
    def _():
        ctx = _dot(ctxraw_s[:], wv_ref[:], ((1,), (1,)))
        ctx_s[:] = ctx
        oppre_s[:] = _dot(ctx, wop_ref[:], ((1,), (1,)))
        seed = ctx[:, None, :] + slotq_ref[:][None, :, :]
        q2 = _dot(seed, wsq_ref[:], ((2,), (1,)))
        q2k_s[:] = _dot(q2, wk_ref[:], ((2,), (0,))) * _RSQRT_H
        g_s[:] = _dot(seed, wg_ref[:], ((2,), (1,)))[:, :, 0] + bg_ref[0, 0]
        bestv_s[:] = jnp.full_like(bestv_s, -jnp.inf)

    @pl.when(is_b)
    def _():
        q2 = q2k_s[pl.ds(b, 1), :, :].reshape(NSLOT, H)
        sc = jnp.concatenate(
            [_dot(q2, lo, ((1,), (1,))), _dot(q2, hi, ((1,), (1,)))], axis=1)
        m = jnp.max(sc, axis=1, keepdims=True)
        e = jnp.exp(sc - m)
        w = e / jnp.sum(e, axis=1, keepdims=True)
        raw2_s[pl.ds(b, 1), :, :] = (
            _dot(w[:, :S // 2], lo, ((1,), (0,)))
            + _dot(w[:, S // 2:], hi, ((1,), (0,))))[None]

    @pl.when(jnp.logical_and(is_b, b < N_KT))
    def _():
        ct = c_ref[:]
        logits = _dot(oppre_s[:], ct, ((1,), (1,)))
        cn = 0.5 * jnp.sum(ct * ct, axis=1)
        val = logits - cn[None, :]
        tmax = jnp.max(val, axis=1, keepdims=True)
        iota = jax.lax.broadcasted_iota(jnp.int32, (B, K_T), 1)
        idx = jnp.min(jnp.where(val >= tmax, iota, K_T), axis=1)
        oh = (iota == idx[:, None]).astype(jnp.float32)
        row = _dot(oh, ct, ((1,), (0,)))
        better = tmax > bestv_s[:]
        bestv_s[:] = jnp.where(better, tmax, bestv_s[:])
        beste_s[:] = jnp.where(better, row, beste_s[:])

    @pl.when(jnp.logical_and(is_b, b == B - 1))
    def _():
        slot_t = _dot(raw2_s[:], wv_ref[:], ((2,), (1,)))
        probs = jax.nn.sigmoid(g_s[:])
        mask = probs >= 0.5
        any_used = jnp.sum(mask.astype(jnp.int32)) > 0
        pmax = jnp.max(probs, axis=1, keepdims=True)
        piota = jax.lax.broadcasted_iota(jnp.int32, (B, NSLOT), 1)
        top = jnp.min(jnp.where(probs >= pmax, piota, NSLOT), axis=1)
        fb = piota == top[:, None]
        mask_f = jnp.where(any_used, mask.astype(jnp.float32),
                           fb.astype(jnp.float32))
        denom = jnp.clip(jnp.sum(mask_f, axis=1, keepdims=True), 1.0, None)
        ssum = jnp.sum(slot_t * mask_f[:, :, None], axis=1) / denom
        msum = jnp.tanh(beste_s[:] + ssum)
        ctx = ctx_s[:]
        stop = (_dot(ctx, ws1_ref[:], ((1,), (1,)))
                + _dot(msum, ws2_ref[:], ((1,), (1,)))
                + bs_ref[0, 0])
        sprob = jax.nn.sigmoid(stop)
        slog_s[pl.ds(s, 1), :] = stop[:, 0][None]
        sprob_s[pl.ds(s, 1), :] = sprob[:, 0][None]
        summary_ref[pl.ds(s, 1), :, :] = msum[None]
        gi = _dot(msum, wih_ref[:], ((1,), (1,))) + bih_ref[:]
        gh_r = gh_s[:, 0, :] + bhh_ref[:, :H]
        gh_z = gh_s[:, 1, :] + bhh_ref[:, H:2 * H]
        gh_n = gh_s[:, 2, :] + bhh_ref[:, 2 * H:]
        r = jax.nn.sigmoid(gi[:, :H] + gh_r)
        z = jax.nn.sigmoid(gi[:, H:2 * H] + gh_z)
        n = jnp.tanh(gi[:, 2 * H:] + r * gh_n)
        state_s[:] = (1.0 - z) * n + z * state_s[:]

    @pl.when(g == 2 * STEPS * B - 1)
    def _():
        stoplog_ref[:] = slog_s[:]
        stopprob_ref[:] = sprob_s[:]
        hits = (sprob_s[:] >= 0.5).astype(jnp.int32)
        siota = jax.lax.broadcasted_iota(jnp.int32, (STEPS, B), 0)
        hmax = jnp.max(hits, axis=0, keepdims=True)
        first = jnp.min(jnp.where(hits >= hmax, siota, STEPS), axis=0,
                        keepdims=True)
        cl = first + 1
        chain_ref[:] = jnp.where(jnp.sum(hits, axis=0, keepdims=True) == 0,
                                 jnp.full_like(cl, STEPS), cl)


def _full(shape):
    return pl.BlockSpec(shape, lambda g: (0,) * len(shape))


@jax.jit
def kernel(logic_hidden, prompt_hidden, codebook_emb, W_init, W_q, W_k, W_v,
           slot_queries, W_slot_q, W_op_pre, W_gate, b_gate, W_stop, b_stop,
           W_ih, W_hh, b_ih, b_hh):
    f32 = jnp.float32
    bg2 = b_gate.reshape(1, 1)
    bs2 = b_stop.reshape(1, 1)
    ws1 = W_stop[:, :H]
    ws2 = W_stop[:, H:]
    bih2 = b_ih.reshape(1, 3 * H)
    bhh2 = b_hh.reshape(1, 3 * H)

    state0 = pl.pallas_call(
        _init_kernel,
        grid=(N_ST,),
        in_specs=[
            pl.BlockSpec((B, S_T, H), lambda i: (0, i, 0)),
            pl.BlockSpec((B, S_T, H), lambda i: (0, i, 0)),
            _full((H, 2 * H)),
        ],
        out_specs=_full((B, H)),
        out_shape=jax.ShapeDtypeStruct((B, H), f32),
        scratch_shapes=[pltpu.VMEM((B, H), f32), pltpu.VMEM((B, H), f32)],
    )(prompt_hidden, logic_hidden, W_init)

    def c_idx(g):
        p = g // B
        b = g % B
        return (jnp.where(p % 2 == 1, jnp.minimum(b, N_KT - 1), N_KT - 1), 0)

    def whh_idx(g):
        p = g // B
        b = g % B
        return (jnp.where(p % 2 == 0, jnp.minimum(b, 2), 2), 0)

    outs = pl.pallas_call(
        _chain_body,
        grid=(2 * STEPS * B,),
        in_specs=[
            _full((B, H)), _full((NSLOT, H)),
            _full((H, H)), _full((H, H)), _full((H, H)), _full((H, H)),
            _full((H, H)),
            _full((1, H)), _full((1, 1)), _full((1, H)), _full((1, H)),
            _full((1, 1)), _full((3 * H, H)),
            pl.BlockSpec((H, H), whh_idx),
            _full((1, 3 * H)), _full((1, 3 * H)),
            pl.BlockSpec((1, S // 2, H), lambda g: (g % B, 0, 0)),
            pl.BlockSpec((1, S // 2, H), lambda g: (g % B, 1, 0)),
            pl.BlockSpec((K_T, H), c_idx),
        ],
        out_specs=[_full((STEPS, B)), _full((STEPS, B)),
                   _full((STEPS, B, H)), _full((1, B))],
        out_shape=[jax.ShapeDtypeStruct((STEPS, B), f32),
                   jax.ShapeDtypeStruct((STEPS, B), f32),
                   jax.ShapeDtypeStruct((STEPS, B, H), f32),
                   jax.ShapeDtypeStruct((1, B), jnp.int32)],
        scratch_shapes=[
            pltpu.VMEM((B, H), f32),
            pltpu.VMEM((B, H), f32),
            pltpu.VMEM((B, H), f32),
            pltpu.VMEM((B, H), f32),
            pltpu.VMEM((B, H), f32),
            pltpu.VMEM((B, NSLOT, H), f32),
            pltpu.VMEM((B, NSLOT), f32),
            pltpu.VMEM((B, NSLOT, H), f32),
            pltpu.VMEM((B, 1), f32),
            pltpu.VMEM((B, H), f32),
            pltpu.VMEM((STEPS, B), f32),
            pltpu.VMEM((STEPS, B), f32),
            pltpu.VMEM((B, 3, H), f32),
        ],
    )(state0, slot_queries, W_q, W_k, W_v, W_op_pre, W_slot_q,
      W_gate, bg2, ws1, ws2, bs2, W_ih, W_hh, bih2, bhh2,
      prompt_hidden, prompt_hidden, codebook_emb)

    slog, sprob, summ, chain = outs
    return (slog.T, sprob.T, jnp.transpose(summ, (1, 0, 2)), chain[0])


# speedup vs baseline: 1.4377x; 1.4377x over previous
import math

import jax
import jax.numpy as jnp
from jax.experimental import pallas as pl
from jax.experimental.pallas import tpu as pltpu

B = 16
S = 2048
H = 768
K = 8192
NSLOT = 9
STEPS = 4

S_T = 128
N_ST = S // S_T
K_T = 512
N_KT = K // K_T

_RSQRT_H = 1.0 / math.sqrt(float(H))


def _init_kernel(p_ref, l_ref, w_init_ref, state_ref, psum, lsum):
    i = pl.program_id(0)

    @pl.when(i == 0)
    def _():
        psum[:] = jnp.zeros_like(psum)
        lsum[:] = jnp.zeros_like(lsum)

    psum[:] += jnp.sum(p_ref[:], axis=1)
    lsum[:] += jnp.sum(l_ref[:], axis=1)

    @pl.when(i == N_ST - 1)
    def _():
        ps = psum[:] * (1.0 / S)
        ls = lsum[:] * (1.0 / S)
        cat = jnp.concatenate([ps, ls], axis=-1)
        state_ref[:] = jnp.tanh(_dot(cat, w_init_ref[:], ((1,), (1,))))


def _dot(a, b, dims):
    return jax.lax.dot_general(a, b, (dims, ((), ())),
                               preferred_element_type=jnp.float32,
                               precision=jax.lax.Precision.HIGHEST)


def _dot_fast(a, b, dims):
    return jax.lax.dot_general(a, b, (dims, ((), ())),
                               preferred_element_type=jnp.float32)


def _chain_body(state0_ref, slotq_ref, wq_ref, wk_ref, wv_ref, wop_ref,
                wsq_ref, wg_ref, bg_ref, ws1_ref, ws2_ref, bs_ref,
                wih_ref, whh_ref, bih_ref, bhh_ref,
                plo_ref, phi_ref, c_ref,
                stoplog_ref, stopprob_ref, summary_ref, chain_ref,
                state_s, q1k_s, ctxraw_s, ctx_s, oppre_s, q2k_s, g_s,
                raw2_s, bestv_s, beste_s, slog_s, sprob_s, gh_s):
    g = pl.program_id(0)
    p = g // B
    b = g % B
    s = p // 2
    is_a = (p % 2) == 0
    is_b = jnp.logical_not(is_a)

    @pl.when(g == 0)
    def _():
        state_s[:] = state0_ref[:]

    lo = plo_ref[0]
    hi = phi_ref[0]

    @pl.when(jnp.logical_and(is_a, b == 0))
    def _():
        sq = _dot(state_s[:], wq_ref[:], ((1,), (1,)))
        q1k_s[:] = _dot(sq, wk_ref[:], ((1,), (0,))) * _RSQRT_H

    @pl.when(jnp.logical_and(is_a, b < 3))
    def _():
        gh_s[:, pl.ds(b, 1), :] = _dot(state_s[:], whh_ref[:],
                                       ((1,), (1,)))[:, None, :]

    @pl.when(is_a)
    def _():
        q = q1k_s[pl.ds(b, 1), :]
        sc = jnp.concatenate(
            [_dot(q, lo, ((1,), (1,))), _dot(q, hi, ((1,), (1,)))], axis=1)
        m = jnp.max(sc, axis=1, keepdims=True)
        e = jnp.exp(sc - m)
        w = e / jnp.sum(e, axis=1, keepdims=True)
        ctxraw_s[pl.ds(b, 1), :] = (
            _dot(w[:, :S // 2], lo, ((1,), (0,)))
            + _dot(w[:, S // 2:], hi, ((1,), (0,))))

    @pl.when(jnp.logical_and(is_a, b == B - 1))
    def _():
        ctx = _dot(ctxraw_s[:], wv_ref[:], ((1,), (1,)))
        ctx_s[:] = ctx
        oppre_s[:] = _dot(ctx, wop_ref[:], ((1,), (1,)))
        seed = ctx[:, None, :] + slotq_ref[:][None, :, :]
        q2 = _dot(seed, wsq_ref[:], ((2,), (1,)))
        q2k_s[:] = _dot(q2, wk_ref[:], ((2,), (0,))) * _RSQRT_H
        g_s[:] = _dot(seed, wg_ref[:], ((2,), (1,)))[:, :, 0] + bg_ref[0, 0]
        bestv_s[:] = jnp.full_like(bestv_s, -jnp.inf)

    @pl.when(is_b)
    def _():
        q2 = q2k_s[pl.ds(b, 1), :, :].reshape(NSLOT, H)
        sc = jnp.concatenate(
            [_dot_fast(q2, lo, ((1,), (1,))),
             _dot_fast(q2, hi, ((1,), (1,)))], axis=1)
        m = jnp.max(sc, axis=1, keepdims=True)
        e = jnp.exp(sc - m)
        w = e / jnp.sum(e, axis=1, keepdims=True)
        raw2_s[pl.ds(b, 1), :, :] = (
            _dot_fast(w[:, :S // 2], lo, ((1,), (0,)))
            + _dot_fast(w[:, S // 2:], hi, ((1,), (0,))))[None]

    @pl.when(jnp.logical_and(is_b, b < N_KT))
    def _():
        ct = c_ref[:]
        logits = _dot(oppre_s[:], ct, ((1,), (1,)))
        cn = 0.5 * jnp.sum(ct * ct, axis=1)
        val = logits - cn[None, :]
        tmax = jnp.max(val, axis=1, keepdims=True)
        iota = jax.lax.broadcasted_iota(jnp.int32, (B, K_T), 1)
        idx = jnp.min(jnp.where(val >= tmax, iota, K_T), axis=1)
        oh = (iota == idx[:, None]).astype(jnp.float32)
        row = _dot(oh, ct, ((1,), (0,)))
        better = tmax > bestv_s[:]
        bestv_s[:] = jnp.where(better, tmax, bestv_s[:])
        beste_s[:] = jnp.where(better, row, beste_s[:])

    @pl.when(jnp.logical_and(is_b, b == B - 1))
    def _():
        slot_t = _dot(raw2_s[:], wv_ref[:], ((2,), (1,)))
        probs = jax.nn.sigmoid(g_s[:])
        mask = probs >= 0.5
        any_used = jnp.sum(mask.astype(jnp.int32)) > 0
        pmax = jnp.max(probs, axis=1, keepdims=True)
        piota = jax.lax.broadcasted_iota(jnp.int32, (B, NSLOT), 1)
        top = jnp.min(jnp.where(probs >= pmax, piota, NSLOT), axis=1)
        fb = piota == top[:, None]
        mask_f = jnp.where(any_used, mask.astype(jnp.float32),
                           fb.astype(jnp.float32))
        denom = jnp.clip(jnp.sum(mask_f, axis=1, keepdims=True), 1.0, None)
        ssum = jnp.sum(slot_t * mask_f[:, :, None], axis=1) / denom
        msum = jnp.tanh(beste_s[:] + ssum)
        ctx = ctx_s[:]
        stop = (_dot(ctx, ws1_ref[:], ((1,), (1,)))
                + _dot(msum, ws2_ref[:], ((1,), (1,)))
                + bs_ref[0, 0])
        sprob = jax.nn.sigmoid(stop)
        slog_s[pl.ds(s, 1), :] = stop[:, 0][None]
        sprob_s[pl.ds(s, 1), :] = sprob[:, 0][None]
        summary_ref[pl.ds(s, 1), :, :] = msum[None]
        gi = _dot(msum, wih_ref[:], ((1,), (1,))) + bih_ref[:]
        gh_r = gh_s[:, 0, :] + bhh_ref[:, :H]
        gh_z = gh_s[:, 1, :] + bhh_ref[:, H:2 * H]
        gh_n = gh_s[:, 2, :] + bhh_ref[:, 2 * H:]
        r = jax.nn.sigmoid(gi[:, :H] + gh_r)
        z = jax.nn.sigmoid(gi[:, H:2 * H] + gh_z)
        n = jnp.tanh(gi[:, 2 * H:] + r * gh_n)
        state_s[:] = (1.0 - z) * n + z * state_s[:]

    @pl.when(g == 2 * STEPS * B - 1)
    def _():
        stoplog_ref[:] = slog_s[:]
        stopprob_ref[:] = sprob_s[:]
        hits = (sprob_s[:] >= 0.5).astype(jnp.int32)
        siota = jax.lax.broadcasted_iota(jnp.int32, (STEPS, B), 0)
        hmax = jnp.max(hits, axis=0, keepdims=True)
        first = jnp.min(jnp.where(hits >= hmax, siota, STEPS), axis=0,
                        keepdims=True)
        cl = first + 1
        chain_ref[:] = jnp.where(jnp.sum(hits, axis=0, keepdims=True) == 0,
                                 jnp.full_like(cl, STEPS), cl)


def _full(shape):
    return pl.BlockSpec(shape, lambda g: (0,) * len(shape))


@jax.jit
def kernel(logic_hidden, prompt_hidden, codebook_emb, W_init, W_q, W_k, W_v,
           slot_queries, W_slot_q, W_op_pre, W_gate, b_gate, W_stop, b_stop,
           W_ih, W_hh, b_ih, b_hh):
    f32 = jnp.float32
    bg2 = b_gate.reshape(1, 1)
    bs2 = b_stop.reshape(1, 1)
    ws1 = W_stop[:, :H]
    ws2 = W_stop[:, H:]
    bih2 = b_ih.reshape(1, 3 * H)
    bhh2 = b_hh.reshape(1, 3 * H)

    state0 = pl.pallas_call(
        _init_kernel,
        grid=(N_ST,),
        in_specs=[
            pl.BlockSpec((B, S_T, H), lambda i: (0, i, 0)),
            pl.BlockSpec((B, S_T, H), lambda i: (0, i, 0)),
            _full((H, 2 * H)),
        ],
        out_specs=_full((B, H)),
        out_shape=jax.ShapeDtypeStruct((B, H), f32),
        scratch_shapes=[pltpu.VMEM((B, H), f32), pltpu.VMEM((B, H), f32)],
    )(prompt_hidden, logic_hidden, W_init)

    def c_idx(g):
        p = g // B
        b = g % B
        return (jnp.where(p % 2 == 1, jnp.minimum(b, N_KT - 1), N_KT - 1), 0)

    def whh_idx(g):
        p = g // B
        b = g % B
        return (jnp.where(p % 2 == 0, jnp.minimum(b, 2), 2), 0)

    outs = pl.pallas_call(
        _chain_body,
        grid=(2 * STEPS * B,),
        in_specs=[
            _full((B, H)), _full((NSLOT, H)),
            _full((H, H)), _full((H, H)), _full((H, H)), _full((H, H)),
            _full((H, H)),
            _full((1, H)), _full((1, 1)), _full((1, H)), _full((1, H)),
            _full((1, 1)), _full((3 * H, H)),
            pl.BlockSpec((H, H), whh_idx),
            _full((1, 3 * H)), _full((1, 3 * H)),
            pl.BlockSpec((1, S // 2, H), lambda g: (g % B, 0, 0)),
            pl.BlockSpec((1, S // 2, H), lambda g: (g % B, 1, 0)),
            pl.BlockSpec((K_T, H), c_idx),
        ],
        out_specs=[_full((STEPS, B)), _full((STEPS, B)),
                   _full((STEPS, B, H)), _full((1, B))],
        out_shape=[jax.ShapeDtypeStruct((STEPS, B), f32),
                   jax.ShapeDtypeStruct((STEPS, B), f32),
                   jax.ShapeDtypeStruct((STEPS, B, H), f32),
                   jax.ShapeDtypeStruct((1, B), jnp.int32)],
        scratch_shapes=[
            pltpu.VMEM((B, H), f32),
            pltpu.VMEM((B, H), f32),
            pltpu.VMEM((B, H), f32),
            pltpu.VMEM((B, H), f32),
            pltpu.VMEM((B, H), f32),
            pltpu.VMEM((B, NSLOT, H), f32),
            pltpu.VMEM((B, NSLOT), f32),
            pltpu.VMEM((B, NSLOT, H), f32),
            pltpu.VMEM((B, 1), f32),
            pltpu.VMEM((B, H), f32),
            pltpu.VMEM((STEPS, B), f32),
            pltpu.VMEM((STEPS, B), f32),
            pltpu.VMEM((B, 3, H), f32),
        ],
    )(state0, slot_queries, W_q, W_k, W_v, W_op_pre, W_slot_q,
      W_gate, bg2, ws1, ws2, bs2, W_ih, W_hh, bih2, bhh2,
      prompt_hidden, prompt_hidden, codebook_emb)

    slog, sprob, summ, chain = outs
    return (slog.T, sprob.T, jnp.transpose(summ, (1, 0, 2)), chain[0])
